# baseline (speedup 1.0000x reference)
import functools

import jax
import jax.numpy as jnp
from jax import lax
from jax.experimental import pallas as pl
from jax.experimental.pallas import tpu as pltpu
from jax.experimental.pallas import tpu_sc as plsc

B, S, H = 4, 2048, 1024
N_SPANS = 2048
WIDTH_DIM = 128
NUM_LABELS = 16

NUM_ROWS = B * N_SPANS
HW = H // 2

NC, NS = 2, 16
NW = NC * NS
PER_W = NUM_ROWS // NW
CHUNK = 32
N_CHUNKS = PER_W // CHUNK


NBUF = 3


def _sc_gather_kernel(hs_hbm, wt_hbm, is_hbm, ie_hbm, iw_hbm,
                      os_hbm, oe_hbm, ow_hbm,
                      isv, iev, iwv,
                      rs0, rs1, rs2, re0, re1, re2, rw0, rw1, rw2,
                      sg0, sg1, sg2, so0, so1, so2):
    wid = lax.axis_index("s") * NC + lax.axis_index("c")
    base = wid * PER_W
    pltpu.sync_copy(is_hbm.at[pl.ds(base, PER_W)], isv)
    pltpu.sync_copy(ie_hbm.at[pl.ds(base, PER_W)], iev)
    pltpu.sync_copy(iw_hbm.at[pl.ds(base, PER_W)], iwv)
    rs, re_, rw = (rs0, rs1, rs2), (re0, re1, re2), (rw0, rw1, rw2)
    semg, semo = (sg0, sg1, sg2), (so0, so1, so2)
    gathers, outs = {}, {}

    def issue_gather(ci):
        s = ci % NBUF
        sl = pl.ds(ci * CHUNK, CHUNK)
        gathers[ci] = (
            pltpu.async_copy(hs_hbm.at[isv.at[sl]], rs[s], semg[s]),
            pltpu.async_copy(hs_hbm.at[iev.at[sl]], re_[s], semg[s]),
            pltpu.async_copy(wt_hbm.at[iwv.at[sl]], rw[s], semg[s]),
        )

    def issue_out(ci):
        s = ci % NBUF
        off = pl.ds(base + ci * CHUNK, CHUNK)
        for c in gathers[ci]:
            c.wait()
        outs[ci] = (
            pltpu.async_copy(rs[s], os_hbm.at[off], semo[s]),
            pltpu.async_copy(re_[s], oe_hbm.at[off], semo[s]),
            pltpu.async_copy(rw[s], ow_hbm.at[off], semo[s]),
        )

    for ci in range(N_CHUNKS):
        if ci >= NBUF:
            for c in outs[ci - NBUF]:
                c.wait()
        issue_gather(ci)
        if ci >= 1:
            issue_out(ci - 1)
    issue_out(N_CHUNKS - 1)
    for ci in range(max(0, N_CHUNKS - NBUF), N_CHUNKS):
        for c in outs[ci]:
            c.wait()


def _sc_gather(hs_bits, wt, idx_s, idx_e, idx_w):
    mesh = plsc.VectorSubcoreMesh(core_axis_name="c", subcore_axis_name="s")
    k = pl.kernel(
        _sc_gather_kernel,
        out_type=(
            jax.ShapeDtypeStruct((NUM_ROWS, HW), jnp.int32),
            jax.ShapeDtypeStruct((NUM_ROWS, HW), jnp.int32),
            jax.ShapeDtypeStruct((NUM_ROWS, WIDTH_DIM), jnp.float32),
        ),
        mesh=mesh,
        scratch_types=(
            [pltpu.VMEM((PER_W,), jnp.int32)] * 3
            + [pltpu.VMEM((CHUNK, HW), jnp.int32)] * (2 * NBUF)
            + [pltpu.VMEM((CHUNK, WIDTH_DIM), jnp.float32)] * NBUF
            + [pltpu.SemaphoreType.DMA] * (2 * NBUF)
        ),
    )
    return k(hs_bits, wt, idx_s, idx_e, idx_w)


BM = 512


def _mlp_block(xs_ref, xe_ref, xw_ref, wa_ref, wb_ref, ww_ref,
               b1_ref, w2_ref, b2_ref, out_ref):
    acc = jnp.dot(xs_ref[...], wa_ref[...], preferred_element_type=jnp.float32)
    acc += jnp.dot(xe_ref[...], wb_ref[...], preferred_element_type=jnp.float32)
    acc += jnp.dot(xw_ref[...].astype(jnp.bfloat16), ww_ref[...],
                   preferred_element_type=jnp.float32)
    acc += b1_ref[...]
    h = jnp.maximum(acc, 0.0).astype(jnp.bfloat16)
    out = jnp.dot(h, w2_ref[...], preferred_element_type=jnp.float32)
    out_ref[...] = out + b2_ref[...]


def _tc_mlp(xs, xe, xw, wa, wb, ww, b1, w2, b2):
    grid = (NUM_ROWS // BM,)
    return pl.pallas_call(
        _mlp_block,
        grid=grid,
        in_specs=[
            pl.BlockSpec((BM, H), lambda i: (i, 0)),
            pl.BlockSpec((BM, H), lambda i: (i, 0)),
            pl.BlockSpec((BM, WIDTH_DIM), lambda i: (i, 0)),
            pl.BlockSpec((H, H), lambda i: (0, 0)),
            pl.BlockSpec((H, H), lambda i: (0, 0)),
            pl.BlockSpec((WIDTH_DIM, H), lambda i: (0, 0)),
            pl.BlockSpec((1, H), lambda i: (0, 0)),
            pl.BlockSpec((H, NUM_LABELS), lambda i: (0, 0)),
            pl.BlockSpec((1, NUM_LABELS), lambda i: (0, 0)),
        ],
        out_specs=pl.BlockSpec((BM, NUM_LABELS), lambda i: (i, 0)),
        out_shape=jax.ShapeDtypeStruct((NUM_ROWS, NUM_LABELS), jnp.float32),
        compiler_params=pltpu.CompilerParams(
            dimension_semantics=("parallel",),
        ),
    )(xs, xe, xw, wa, wb, ww, b1, w2, b2)


def kernel(hidden_states, spans, width_table, W1, b1, W2, b2):
    hs_bf = hidden_states.astype(jnp.bfloat16).reshape(B * S, H)
    hs_bits = lax.bitcast_convert_type(
        hs_bf.reshape(B * S, HW, 2), jnp.int32).reshape(B * S, HW)

    offs = (jnp.arange(B, dtype=jnp.int32) * S)[:, None]
    idx_s = (spans[:, :, 0] + offs).reshape(NUM_ROWS)
    idx_e = (spans[:, :, 1] + offs).reshape(NUM_ROWS)
    idx_w = spans[:, :, 2].reshape(NUM_ROWS)

    gs_bits, ge_bits, gw = _sc_gather(hs_bits, width_table, idx_s, idx_e, idx_w)
    gs = lax.bitcast_convert_type(gs_bits, jnp.bfloat16).reshape(NUM_ROWS, H)
    ge = lax.bitcast_convert_type(ge_bits, jnp.bfloat16).reshape(NUM_ROWS, H)

    wa = W1[:H].astype(jnp.bfloat16)
    wb = W1[H:2 * H].astype(jnp.bfloat16)
    ww = W1[2 * H:].astype(jnp.bfloat16)
    b1r = b1.reshape(1, H)
    w2 = W2.astype(jnp.bfloat16)
    b2r = b2.reshape(1, NUM_LABELS)

    logits = _tc_mlp(gs, ge, gw, wa, wb, ww, b1r, w2, b2r)
    return logits.reshape(B, N_SPANS, NUM_LABELS)


# speedup vs baseline: 3.9008x; 3.9008x over previous
import functools

import jax
import jax.numpy as jnp
from jax import lax
from jax.experimental import pallas as pl
from jax.experimental.pallas import tpu as pltpu
from jax.experimental.pallas import tpu_sc as plsc

B, S, H = 4, 2048, 1024
N_SPANS = 2048
WIDTH_DIM = 128
NUM_LABELS = 16

NUM_ROWS = B * N_SPANS
HW = H // 2

NC, NS = 2, 16
NW = NC * NS
PER_W = NUM_ROWS // NW
CHUNK = 16
N_CHUNKS = PER_W // CHUNK


NBUF = 3


def _sc_gather_kernel(hs_hbm, wt_hbm, is_hbm, ie_hbm, iw_hbm,
                      os_hbm, oe_hbm, ow_hbm,
                      isv, iev, iwv,
                      rs0, rs1, rs2, re0, re1, re2, rw0, rw1, rw2,
                      sg0, sg1, sg2, so0, so1, so2):
    wid = lax.axis_index("s") * NC + lax.axis_index("c")
    base = wid * PER_W
    pltpu.sync_copy(is_hbm.at[pl.ds(base, PER_W)], isv)
    pltpu.sync_copy(ie_hbm.at[pl.ds(base, PER_W)], iev)
    pltpu.sync_copy(iw_hbm.at[pl.ds(base, PER_W)], iwv)
    rs, re_, rw = (rs0, rs1, rs2), (re0, re1, re2), (rw0, rw1, rw2)
    semg, semo = (sg0, sg1, sg2), (so0, so1, so2)
    gathers, outs = {}, {}

    def issue_gather(ci):
        s = ci % NBUF
        sl = pl.ds(ci * CHUNK, CHUNK)
        gathers[ci] = (
            pltpu.async_copy(hs_hbm.at[isv.at[sl]], rs[s], semg[s]),
            pltpu.async_copy(hs_hbm.at[iev.at[sl]], re_[s], semg[s]),
            pltpu.async_copy(wt_hbm.at[iwv.at[sl]], rw[s], semg[s]),
        )

    def issue_out(ci):
        s = ci % NBUF
        off = pl.ds(base + ci * CHUNK, CHUNK)
        for c in gathers[ci]:
            c.wait()
        outs[ci] = (
            pltpu.async_copy(rs[s], os_hbm.at[off], semo[s]),
            pltpu.async_copy(re_[s], oe_hbm.at[off], semo[s]),
            pltpu.async_copy(rw[s], ow_hbm.at[off], semo[s]),
        )

    for ci in range(N_CHUNKS):
        if ci >= NBUF:
            for c in outs[ci - NBUF]:
                c.wait()
        issue_gather(ci)
        if ci >= 1:
            issue_out(ci - 1)
    issue_out(N_CHUNKS - 1)
    for ci in range(max(0, N_CHUNKS - NBUF), N_CHUNKS):
        for c in outs[ci]:
            c.wait()


def _sc_gather(hs_bits, wt, idx_s, idx_e, idx_w):
    mesh = plsc.VectorSubcoreMesh(core_axis_name="c", subcore_axis_name="s")
    k = pl.kernel(
        _sc_gather_kernel,
        out_type=(
            jax.ShapeDtypeStruct((NUM_ROWS, H), jnp.float32),
            jax.ShapeDtypeStruct((NUM_ROWS, H), jnp.float32),
            jax.ShapeDtypeStruct((NUM_ROWS, WIDTH_DIM), jnp.float32),
        ),
        mesh=mesh,
        scratch_types=(
            [pltpu.VMEM((PER_W,), jnp.int32)] * 3
            + [pltpu.VMEM((CHUNK, H), jnp.float32)] * (2 * NBUF)
            + [pltpu.VMEM((CHUNK, WIDTH_DIM), jnp.float32)] * NBUF
            + [pltpu.SemaphoreType.DMA] * (2 * NBUF)
        ),
    )
    return k(hs_bits, wt, idx_s, idx_e, idx_w)


BM = 512


def _mlp_block(xs_ref, xe_ref, xw_ref, wa_ref, wb_ref, ww_ref,
               b1_ref, w2_ref, b2_ref, out_ref):
    acc = jnp.dot(xs_ref[...].astype(jnp.bfloat16), wa_ref[...],
                  preferred_element_type=jnp.float32)
    acc += jnp.dot(xe_ref[...].astype(jnp.bfloat16), wb_ref[...],
                   preferred_element_type=jnp.float32)
    acc += jnp.dot(xw_ref[...].astype(jnp.bfloat16), ww_ref[...],
                   preferred_element_type=jnp.float32)
    acc += b1_ref[...]
    h = jnp.maximum(acc, 0.0).astype(jnp.bfloat16)
    out = jnp.dot(h, w2_ref[...], preferred_element_type=jnp.float32)
    out_ref[...] = out + b2_ref[...]


def _tc_mlp(xs, xe, xw, wa, wb, ww, b1, w2, b2):
    grid = (NUM_ROWS // BM,)
    return pl.pallas_call(
        _mlp_block,
        grid=grid,
        in_specs=[
            pl.BlockSpec((BM, H), lambda i: (i, 0)),
            pl.BlockSpec((BM, H), lambda i: (i, 0)),
            pl.BlockSpec((BM, WIDTH_DIM), lambda i: (i, 0)),
            pl.BlockSpec((H, H), lambda i: (0, 0)),
            pl.BlockSpec((H, H), lambda i: (0, 0)),
            pl.BlockSpec((WIDTH_DIM, H), lambda i: (0, 0)),
            pl.BlockSpec((1, H), lambda i: (0, 0)),
            pl.BlockSpec((H, NUM_LABELS), lambda i: (0, 0)),
            pl.BlockSpec((1, NUM_LABELS), lambda i: (0, 0)),
        ],
        out_specs=pl.BlockSpec((BM, NUM_LABELS), lambda i: (i, 0)),
        out_shape=jax.ShapeDtypeStruct((NUM_ROWS, NUM_LABELS), jnp.float32),
        compiler_params=pltpu.CompilerParams(
            dimension_semantics=("parallel",),
        ),
    )(xs, xe, xw, wa, wb, ww, b1, w2, b2)


def kernel(hidden_states, spans, width_table, W1, b1, W2, b2):
    hs_flat = hidden_states.reshape(B * S, H)

    offs = (jnp.arange(B, dtype=jnp.int32) * S)[:, None]
    idx_s = (spans[:, :, 0] + offs).reshape(NUM_ROWS)
    idx_e = (spans[:, :, 1] + offs).reshape(NUM_ROWS)
    idx_w = spans[:, :, 2].reshape(NUM_ROWS)

    gs, ge, gw = _sc_gather(hs_flat, width_table, idx_s, idx_e, idx_w)

    wa = W1[:H].astype(jnp.bfloat16)
    wb = W1[H:2 * H].astype(jnp.bfloat16)
    ww = W1[2 * H:].astype(jnp.bfloat16)
    b1r = b1.reshape(1, H)
    w2 = W2.astype(jnp.bfloat16)
    b2r = b2.reshape(1, NUM_LABELS)

    logits = _tc_mlp(gs, ge, gw, wa, wb, ww, b1r, w2, b2r)
    return logits.reshape(B, N_SPANS, NUM_LABELS)


# speedup vs baseline: 3.9673x; 1.0171x over previous
import jax
import jax.numpy as jnp
from jax import lax
from jax.experimental import pallas as pl
from jax.experimental.pallas import tpu as pltpu
from jax.experimental.pallas import tpu_sc as plsc

B, S, H = 4, 2048, 1024
N_SPANS = 2048
WIDTH_DIM = 128
NUM_LABELS = 16

NUM_ROWS = B * N_SPANS
NSPLIT = 4
ROWS_PER_SPLIT = NUM_ROWS // NSPLIT

NC, NS = 2, 16
NW = NC * NS
CHUNK = 16
NBUF = 3


def _make_sc_gather(nrows):
    per_w = nrows // NW
    n_chunks = per_w // CHUNK

    def body(hs_hbm, wt_hbm, is_hbm, ie_hbm, iw_hbm,
             os_hbm, oe_hbm, ow_hbm,
             isv, iev, iwv,
             rs0, rs1, rs2, re0, re1, re2, rw0, rw1, rw2,
             sg0, sg1, sg2, so0, so1, so2):
        wid = lax.axis_index("s") * NC + lax.axis_index("c")
        base = wid * per_w
        pltpu.sync_copy(is_hbm.at[pl.ds(base, per_w)], isv)
        pltpu.sync_copy(ie_hbm.at[pl.ds(base, per_w)], iev)
        pltpu.sync_copy(iw_hbm.at[pl.ds(base, per_w)], iwv)
        rs, re_, rw = (rs0, rs1, rs2), (re0, re1, re2), (rw0, rw1, rw2)
        semg, semo = (sg0, sg1, sg2), (so0, so1, so2)
        gathers, outs = {}, {}

        def issue_gather(ci):
            s = ci % NBUF
            sl = pl.ds(ci * CHUNK, CHUNK)
            gathers[ci] = (
                pltpu.async_copy(hs_hbm.at[isv.at[sl]], rs[s], semg[s]),
                pltpu.async_copy(hs_hbm.at[iev.at[sl]], re_[s], semg[s]),
                pltpu.async_copy(wt_hbm.at[iwv.at[sl]], rw[s], semg[s]),
            )

        def issue_out(ci):
            s = ci % NBUF
            off = pl.ds(base + ci * CHUNK, CHUNK)
            for c in gathers[ci]:
                c.wait()
            outs[ci] = (
                pltpu.async_copy(rs[s], os_hbm.at[off], semo[s]),
                pltpu.async_copy(re_[s], oe_hbm.at[off], semo[s]),
                pltpu.async_copy(rw[s], ow_hbm.at[off], semo[s]),
            )

        for ci in range(n_chunks):
            if ci >= NBUF:
                for c in outs[ci - NBUF]:
                    c.wait()
            issue_gather(ci)
            if ci >= 1:
                issue_out(ci - 1)
        issue_out(n_chunks - 1)
        for ci in range(max(0, n_chunks - NBUF), n_chunks):
            for c in outs[ci]:
                c.wait()

    mesh = plsc.VectorSubcoreMesh(core_axis_name="c", subcore_axis_name="s")
    return pl.kernel(
        body,
        out_type=(
            jax.ShapeDtypeStruct((nrows, H), jnp.float32),
            jax.ShapeDtypeStruct((nrows, H), jnp.float32),
            jax.ShapeDtypeStruct((nrows, WIDTH_DIM), jnp.float32),
        ),
        mesh=mesh,
        scratch_types=(
            [pltpu.VMEM((per_w,), jnp.int32)] * 3
            + [pltpu.VMEM((CHUNK, H), jnp.float32)] * (2 * NBUF)
            + [pltpu.VMEM((CHUNK, WIDTH_DIM), jnp.float32)] * NBUF
            + [pltpu.SemaphoreType.DMA] * (2 * NBUF)
        ),
    )


BM = 512


def _mlp_block(xs_ref, xe_ref, xw_ref, wa_ref, wb_ref, ww_ref,
               b1_ref, w2_ref, b2_ref, out_ref):
    acc = jnp.dot(xs_ref[...].astype(jnp.bfloat16), wa_ref[...],
                  preferred_element_type=jnp.float32)
    acc += jnp.dot(xe_ref[...].astype(jnp.bfloat16), wb_ref[...],
                   preferred_element_type=jnp.float32)
    acc += jnp.dot(xw_ref[...].astype(jnp.bfloat16), ww_ref[...],
                   preferred_element_type=jnp.float32)
    acc += b1_ref[...]
    h = jnp.maximum(acc, 0.0).astype(jnp.bfloat16)
    out = jnp.dot(h, w2_ref[...], preferred_element_type=jnp.float32)
    out_ref[...] = out + b2_ref[...]


def _tc_mlp(xs, xe, xw, wa, wb, ww, b1, w2, b2):
    nrows = xs.shape[0]
    grid = (nrows // BM,)
    return pl.pallas_call(
        _mlp_block,
        grid=grid,
        in_specs=[
            pl.BlockSpec((BM, H), lambda i: (i, 0)),
            pl.BlockSpec((BM, H), lambda i: (i, 0)),
            pl.BlockSpec((BM, WIDTH_DIM), lambda i: (i, 0)),
            pl.BlockSpec((H, H), lambda i: (0, 0)),
            pl.BlockSpec((H, H), lambda i: (0, 0)),
            pl.BlockSpec((WIDTH_DIM, H), lambda i: (0, 0)),
            pl.BlockSpec((1, H), lambda i: (0, 0)),
            pl.BlockSpec((H, NUM_LABELS), lambda i: (0, 0)),
            pl.BlockSpec((1, NUM_LABELS), lambda i: (0, 0)),
        ],
        out_specs=pl.BlockSpec((BM, NUM_LABELS), lambda i: (i, 0)),
        out_shape=jax.ShapeDtypeStruct((nrows, NUM_LABELS), jnp.float32),
        compiler_params=pltpu.CompilerParams(
            dimension_semantics=("parallel",),
        ),
    )(xs, xe, xw, wa, wb, ww, b1, w2, b2)


def kernel(hidden_states, spans, width_table, W1, b1, W2, b2):
    hs_flat = hidden_states.reshape(B * S, H)

    offs = (jnp.arange(B, dtype=jnp.int32) * S)[:, None]
    idx_s = (spans[:, :, 0] + offs).reshape(NUM_ROWS)
    idx_e = (spans[:, :, 1] + offs).reshape(NUM_ROWS)
    idx_w = spans[:, :, 2].reshape(NUM_ROWS)

    wa = W1[:H].astype(jnp.bfloat16)
    wb = W1[H:2 * H].astype(jnp.bfloat16)
    ww = W1[2 * H:].astype(jnp.bfloat16)
    b1r = b1.reshape(1, H)
    w2 = W2.astype(jnp.bfloat16)
    b2r = b2.reshape(1, NUM_LABELS)

    sc_gather = _make_sc_gather(ROWS_PER_SPLIT)
    parts = []
    for c in range(NSPLIT):
        sl = slice(c * ROWS_PER_SPLIT, (c + 1) * ROWS_PER_SPLIT)
        gs, ge, gw = sc_gather(hs_flat, width_table,
                               idx_s[sl], idx_e[sl], idx_w[sl])
        parts.append(_tc_mlp(gs, ge, gw, wa, wb, ww, b1r, w2, b2r))

    logits = jnp.concatenate(parts, axis=0)
    return logits.reshape(B, N_SPANS, NUM_LABELS)


# speedup vs baseline: 4.0931x; 1.0317x over previous
import jax
import jax.numpy as jnp
from jax import lax
from jax.experimental import pallas as pl
from jax.experimental.pallas import tpu as pltpu
from jax.experimental.pallas import tpu_sc as plsc

B, S, H = 4, 2048, 1024
N_SPANS = 2048
WIDTH_DIM = 128
NUM_LABELS = 16

NUM_ROWS = B * N_SPANS
NSPLIT = 2
ROWS_PER_SPLIT = NUM_ROWS // NSPLIT

NC, NS = 2, 16
NW = NC * NS
CHUNK = 16
NBUF = 3


def _make_sc_gather(nrows):
    per_w = nrows // NW
    n_chunks = per_w // CHUNK

    def body(hs_hbm, wt_hbm, is_hbm, ie_hbm, iw_hbm,
             os_hbm, oe_hbm, ow_hbm,
             isv, iev, iwv,
             rs0, rs1, rs2, re0, re1, re2, rw0, rw1, rw2,
             sg0, sg1, sg2, so0, so1, so2):
        wid = lax.axis_index("s") * NC + lax.axis_index("c")
        base = wid * per_w
        pltpu.sync_copy(is_hbm.at[pl.ds(base, per_w)], isv)
        pltpu.sync_copy(ie_hbm.at[pl.ds(base, per_w)], iev)
        pltpu.sync_copy(iw_hbm.at[pl.ds(base, per_w)], iwv)
        rs, re_, rw = (rs0, rs1, rs2), (re0, re1, re2), (rw0, rw1, rw2)
        semg, semo = (sg0, sg1, sg2), (so0, so1, so2)
        gathers, outs = {}, {}

        def issue_gather(ci):
            s = ci % NBUF
            sl = pl.ds(ci * CHUNK, CHUNK)
            gathers[ci] = (
                pltpu.async_copy(hs_hbm.at[isv.at[sl]], rs[s], semg[s]),
                pltpu.async_copy(hs_hbm.at[iev.at[sl]], re_[s], semg[s]),
                pltpu.async_copy(wt_hbm.at[iwv.at[sl]], rw[s], semg[s]),
            )

        def issue_out(ci):
            s = ci % NBUF
            off = pl.ds(base + ci * CHUNK, CHUNK)
            for c in gathers[ci]:
                c.wait()
            outs[ci] = (
                pltpu.async_copy(rs[s], os_hbm.at[off], semo[s]),
                pltpu.async_copy(re_[s], oe_hbm.at[off], semo[s]),
                pltpu.async_copy(rw[s], ow_hbm.at[off], semo[s]),
            )

        for ci in range(n_chunks):
            if ci >= NBUF:
                for c in outs[ci - NBUF]:
                    c.wait()
            issue_gather(ci)
            if ci >= 1:
                issue_out(ci - 1)
        issue_out(n_chunks - 1)
        for ci in range(max(0, n_chunks - NBUF), n_chunks):
            for c in outs[ci]:
                c.wait()

    mesh = plsc.VectorSubcoreMesh(core_axis_name="c", subcore_axis_name="s")
    return pl.kernel(
        body,
        out_type=(
            jax.ShapeDtypeStruct((nrows, H), jnp.float32),
            jax.ShapeDtypeStruct((nrows, H), jnp.float32),
            jax.ShapeDtypeStruct((nrows, WIDTH_DIM), jnp.float32),
        ),
        mesh=mesh,
        scratch_types=(
            [pltpu.VMEM((per_w,), jnp.int32)] * 3
            + [pltpu.VMEM((CHUNK, H), jnp.float32)] * (2 * NBUF)
            + [pltpu.VMEM((CHUNK, WIDTH_DIM), jnp.float32)] * NBUF
            + [pltpu.SemaphoreType.DMA] * (2 * NBUF)
        ),
    )


BM = 512


def _mlp_block(xs_ref, xe_ref, xw_ref, wa_ref, wb_ref, ww_ref,
               b1_ref, w2_ref, b2_ref, out_ref):
    acc = jnp.dot(xs_ref[...].astype(jnp.bfloat16), wa_ref[...],
                  preferred_element_type=jnp.float32)
    acc += jnp.dot(xe_ref[...].astype(jnp.bfloat16), wb_ref[...],
                   preferred_element_type=jnp.float32)
    acc += jnp.dot(xw_ref[...].astype(jnp.bfloat16), ww_ref[...],
                   preferred_element_type=jnp.float32)
    acc += b1_ref[...]
    h = jnp.maximum(acc, 0.0).astype(jnp.bfloat16)
    out = jnp.dot(h, w2_ref[...], preferred_element_type=jnp.float32)
    out_ref[...] = out + b2_ref[...]


def _tc_mlp(xs, xe, xw, wa, wb, ww, b1, w2, b2):
    nrows = xs.shape[0]
    grid = (nrows // BM,)
    return pl.pallas_call(
        _mlp_block,
        grid=grid,
        in_specs=[
            pl.BlockSpec((BM, H), lambda i: (i, 0)),
            pl.BlockSpec((BM, H), lambda i: (i, 0)),
            pl.BlockSpec((BM, WIDTH_DIM), lambda i: (i, 0)),
            pl.BlockSpec((H, H), lambda i: (0, 0)),
            pl.BlockSpec((H, H), lambda i: (0, 0)),
            pl.BlockSpec((WIDTH_DIM, H), lambda i: (0, 0)),
            pl.BlockSpec((1, H), lambda i: (0, 0)),
            pl.BlockSpec((H, NUM_LABELS), lambda i: (0, 0)),
            pl.BlockSpec((1, NUM_LABELS), lambda i: (0, 0)),
        ],
        out_specs=pl.BlockSpec((BM, NUM_LABELS), lambda i: (i, 0)),
        out_shape=jax.ShapeDtypeStruct((nrows, NUM_LABELS), jnp.float32),
        compiler_params=pltpu.CompilerParams(
            dimension_semantics=("parallel",),
        ),
    )(xs, xe, xw, wa, wb, ww, b1, w2, b2)


def kernel(hidden_states, spans, width_table, W1, b1, W2, b2):
    hs_flat = hidden_states.reshape(B * S, H)

    offs = (jnp.arange(B, dtype=jnp.int32) * S)[:, None]
    idx_s = (spans[:, :, 0] + offs).reshape(NUM_ROWS)
    idx_e = (spans[:, :, 1] + offs).reshape(NUM_ROWS)
    idx_w = spans[:, :, 2].reshape(NUM_ROWS)

    wa = W1[:H].astype(jnp.bfloat16)
    wb = W1[H:2 * H].astype(jnp.bfloat16)
    ww = W1[2 * H:].astype(jnp.bfloat16)
    b1r = b1.reshape(1, H)
    w2 = W2.astype(jnp.bfloat16)
    b2r = b2.reshape(1, NUM_LABELS)

    sc_gather = _make_sc_gather(ROWS_PER_SPLIT)
    parts = []
    for c in range(NSPLIT):
        sl = slice(c * ROWS_PER_SPLIT, (c + 1) * ROWS_PER_SPLIT)
        gs, ge, gw = sc_gather(hs_flat, width_table,
                               idx_s[sl], idx_e[sl], idx_w[sl])
        parts.append(_tc_mlp(gs, ge, gw, wa, wb, ww, b1r, w2, b2r))

    logits = jnp.concatenate(parts, axis=0)
    return logits.reshape(B, N_SPANS, NUM_LABELS)


# speedup vs baseline: 4.1308x; 1.0092x over previous
import jax
import jax.numpy as jnp
from jax import lax
from jax.experimental import pallas as pl
from jax.experimental.pallas import tpu as pltpu
from jax.experimental.pallas import tpu_sc as plsc

B, S, H = 4, 2048, 1024
N_SPANS = 2048
WIDTH_DIM = 128
NUM_LABELS = 16

NUM_ROWS = B * N_SPANS
NSPLIT = 2
ROWS_PER_SPLIT = NUM_ROWS // NSPLIT

NC, NS = 2, 16
NW = NC * NS
CHUNK = 16
NBUF = 3


def _make_sc_gather(nrows):
    per_w = nrows // NW
    n_chunks = per_w // CHUNK

    def body(hs_hbm, wt_hbm, is_hbm, ie_hbm, iw_hbm,
             os_hbm, oe_hbm, ow_hbm,
             is0, is1, is2, ie0, ie1, ie2, iw0, iw1, iw2,
             rs0, rs1, rs2, re0, re1, re2, rw0, rw1, rw2,
             *sems):
        wid = lax.axis_index("s") * NC + lax.axis_index("c")
        base = wid * per_w
        isb, ieb, iwb = (is0, is1, is2), (ie0, ie1, ie2), (iw0, iw1, iw2)
        rs, re_, rw = (rs0, rs1, rs2), (re0, re1, re2), (rw0, rw1, rw2)
        semi = [sems[3 * s:3 * s + 3] for s in range(NBUF)]
        semg = [sems[3 * NBUF + 3 * s:3 * NBUF + 3 * s + 3]
                for s in range(NBUF)]
        semo = [sems[6 * NBUF + 3 * s:6 * NBUF + 3 * s + 3]
                for s in range(NBUF)]
        idxs, gathers, outs = {}, {}, {}

        def issue_idx(ci):
            s = ci % NBUF
            off = pl.ds(base + ci * CHUNK, CHUNK)
            idxs[ci] = (
                pltpu.async_copy(is_hbm.at[off], isb[s], semi[s][0]),
                pltpu.async_copy(ie_hbm.at[off], ieb[s], semi[s][1]),
                pltpu.async_copy(iw_hbm.at[off], iwb[s], semi[s][2]),
            )

        def issue_gather(ci):
            s = ci % NBUF
            for c in idxs[ci]:
                c.wait()
            gathers[ci] = (
                pltpu.async_copy(hs_hbm.at[isb[s]], rs[s], semg[s][0]),
                pltpu.async_copy(hs_hbm.at[ieb[s]], re_[s], semg[s][1]),
                pltpu.async_copy(wt_hbm.at[iwb[s]], rw[s], semg[s][2]),
            )

        def issue_out(ci):
            s = ci % NBUF
            off = pl.ds(base + ci * CHUNK, CHUNK)
            for c in gathers[ci]:
                c.wait()
            outs[ci] = (
                pltpu.async_copy(rs[s], os_hbm.at[off], semo[s][0]),
                pltpu.async_copy(re_[s], oe_hbm.at[off], semo[s][1]),
                pltpu.async_copy(rw[s], ow_hbm.at[off], semo[s][2]),
            )

        issue_idx(0)
        for ci in range(n_chunks):
            if ci + 1 < n_chunks:
                issue_idx(ci + 1)
            if ci >= NBUF:
                for c in outs[ci - NBUF]:
                    c.wait()
            issue_gather(ci)
            if ci >= 1:
                issue_out(ci - 1)
        issue_out(n_chunks - 1)
        for ci in range(max(0, n_chunks - NBUF), n_chunks):
            for c in outs[ci]:
                c.wait()

    mesh = plsc.VectorSubcoreMesh(core_axis_name="c", subcore_axis_name="s")
    return pl.kernel(
        body,
        out_type=(
            jax.ShapeDtypeStruct((nrows, H), jnp.float32),
            jax.ShapeDtypeStruct((nrows, H), jnp.float32),
            jax.ShapeDtypeStruct((nrows, WIDTH_DIM), jnp.float32),
        ),
        mesh=mesh,
        scratch_types=(
            [pltpu.VMEM((CHUNK,), jnp.int32)] * (3 * NBUF)
            + [pltpu.VMEM((CHUNK, H), jnp.float32)] * (2 * NBUF)
            + [pltpu.VMEM((CHUNK, WIDTH_DIM), jnp.float32)] * NBUF
            + [pltpu.SemaphoreType.DMA] * (9 * NBUF)
        ),
    )


BM = 512


def _mlp_block(xs_ref, xe_ref, xw_ref, wa_ref, wb_ref, ww_ref,
               b1_ref, w2_ref, b2_ref, out_ref):
    acc = jnp.dot(xs_ref[...].astype(jnp.bfloat16), wa_ref[...],
                  preferred_element_type=jnp.float32)
    acc += jnp.dot(xe_ref[...].astype(jnp.bfloat16), wb_ref[...],
                   preferred_element_type=jnp.float32)
    acc += jnp.dot(xw_ref[...].astype(jnp.bfloat16), ww_ref[...],
                   preferred_element_type=jnp.float32)
    acc += b1_ref[...]
    h = jnp.maximum(acc, 0.0).astype(jnp.bfloat16)
    out = jnp.dot(h, w2_ref[...], preferred_element_type=jnp.float32)
    out_ref[...] = out + b2_ref[...]


def _tc_mlp(xs, xe, xw, wa, wb, ww, b1, w2, b2):
    nrows = xs.shape[0]
    grid = (nrows // BM,)
    return pl.pallas_call(
        _mlp_block,
        grid=grid,
        in_specs=[
            pl.BlockSpec((BM, H), lambda i: (i, 0)),
            pl.BlockSpec((BM, H), lambda i: (i, 0)),
            pl.BlockSpec((BM, WIDTH_DIM), lambda i: (i, 0)),
            pl.BlockSpec((H, H), lambda i: (0, 0)),
            pl.BlockSpec((H, H), lambda i: (0, 0)),
            pl.BlockSpec((WIDTH_DIM, H), lambda i: (0, 0)),
            pl.BlockSpec((1, H), lambda i: (0, 0)),
            pl.BlockSpec((H, NUM_LABELS), lambda i: (0, 0)),
            pl.BlockSpec((1, NUM_LABELS), lambda i: (0, 0)),
        ],
        out_specs=pl.BlockSpec((BM, NUM_LABELS), lambda i: (i, 0)),
        out_shape=jax.ShapeDtypeStruct((nrows, NUM_LABELS), jnp.float32),
        compiler_params=pltpu.CompilerParams(
            dimension_semantics=("parallel",),
        ),
    )(xs, xe, xw, wa, wb, ww, b1, w2, b2)


def kernel(hidden_states, spans, width_table, W1, b1, W2, b2):
    hs_flat = hidden_states.reshape(B * S, H)

    offs = (jnp.arange(B, dtype=jnp.int32) * S)[:, None]
    idx_s = (spans[:, :, 0] + offs).reshape(NUM_ROWS)
    idx_e = (spans[:, :, 1] + offs).reshape(NUM_ROWS)
    idx_w = spans[:, :, 2].reshape(NUM_ROWS)

    wa = W1[:H].astype(jnp.bfloat16)
    wb = W1[H:2 * H].astype(jnp.bfloat16)
    ww = W1[2 * H:].astype(jnp.bfloat16)
    b1r = b1.reshape(1, H)
    w2 = W2.astype(jnp.bfloat16)
    b2r = b2.reshape(1, NUM_LABELS)

    sc_gather = _make_sc_gather(ROWS_PER_SPLIT)
    parts = []
    for c in range(NSPLIT):
        sl = slice(c * ROWS_PER_SPLIT, (c + 1) * ROWS_PER_SPLIT)
        gs, ge, gw = sc_gather(hs_flat, width_table,
                               idx_s[sl], idx_e[sl], idx_w[sl])
        parts.append(_tc_mlp(gs, ge, gw, wa, wb, ww, b1r, w2, b2r))

    logits = jnp.concatenate(parts, axis=0)
    return logits.reshape(B, N_SPANS, NUM_LABELS)


# speedup vs baseline: 4.1343x; 1.0008x over previous
import jax
import jax.numpy as jnp
from jax import lax
from jax.experimental import pallas as pl
from jax.experimental.pallas import tpu as pltpu
from jax.experimental.pallas import tpu_sc as plsc

B, S, H = 4, 2048, 1024
N_SPANS = 2048
WIDTH_DIM = 128
NUM_LABELS = 16

NUM_ROWS = B * N_SPANS
NSPLIT = 2
ROWS_PER_SPLIT = NUM_ROWS // NSPLIT

NC, NS = 2, 16
NW = NC * NS
CHUNK = 16
NBUF = 3


def _make_sc_gather(nrows):
    per_w = nrows // NW
    n_chunks = per_w // CHUNK

    def body(hs_hbm, wt_hbm, is_hbm, ie_hbm, iw_hbm,
             os_hbm, oe_hbm, ow_hbm,
             is0, is1, is2, ie0, ie1, ie2, iw0, iw1, iw2,
             rs0, rs1, rs2, re0, re1, re2, rw0, rw1, rw2,
             *sems):
        wid = lax.axis_index("s") * NC + lax.axis_index("c")
        base = wid * per_w
        isb, ieb, iwb = (is0, is1, is2), (ie0, ie1, ie2), (iw0, iw1, iw2)
        rs, re_, rw = (rs0, rs1, rs2), (re0, re1, re2), (rw0, rw1, rw2)
        semi = [sems[3 * s:3 * s + 3] for s in range(NBUF)]
        semg = [sems[3 * NBUF + 3 * s:3 * NBUF + 3 * s + 3]
                for s in range(NBUF)]
        semo = [sems[6 * NBUF + 3 * s:6 * NBUF + 3 * s + 3]
                for s in range(NBUF)]
        idxs, gathers, outs = {}, {}, {}

        def issue_idx(ci):
            s = ci % NBUF
            off = pl.ds(base + ci * CHUNK, CHUNK)
            idxs[ci] = (
                pltpu.async_copy(is_hbm.at[off], isb[s], semi[s][0]),
                pltpu.async_copy(ie_hbm.at[off], ieb[s], semi[s][1]),
                pltpu.async_copy(iw_hbm.at[off], iwb[s], semi[s][2]),
            )

        def issue_gather(ci):
            s = ci % NBUF
            for c in idxs[ci]:
                c.wait()
            gathers[ci] = (
                pltpu.async_copy(hs_hbm.at[isb[s]], rs[s], semg[s][0]),
                pltpu.async_copy(hs_hbm.at[ieb[s]], re_[s], semg[s][1]),
                pltpu.async_copy(wt_hbm.at[iwb[s]], rw[s], semg[s][2]),
            )

        def issue_out(ci):
            s = ci % NBUF
            off = pl.ds(base + ci * CHUNK, CHUNK)
            for c in gathers[ci]:
                c.wait()
            outs[ci] = (
                pltpu.async_copy(rs[s], os_hbm.at[off], semo[s][0]),
                pltpu.async_copy(re_[s], oe_hbm.at[off], semo[s][1]),
                pltpu.async_copy(rw[s], ow_hbm.at[off], semo[s][2]),
            )

        issue_idx(0)
        for ci in range(n_chunks):
            if ci + 1 < n_chunks:
                issue_idx(ci + 1)
            if ci >= NBUF:
                for c in outs[ci - NBUF]:
                    c.wait()
            issue_gather(ci)
            if ci >= 1:
                issue_out(ci - 1)
        issue_out(n_chunks - 1)
        for ci in range(max(0, n_chunks - NBUF), n_chunks):
            for c in outs[ci]:
                c.wait()

    mesh = plsc.VectorSubcoreMesh(core_axis_name="c", subcore_axis_name="s")
    return pl.kernel(
        body,
        out_type=(
            jax.ShapeDtypeStruct((nrows, H), jnp.float32),
            jax.ShapeDtypeStruct((nrows, H), jnp.float32),
            jax.ShapeDtypeStruct((nrows, WIDTH_DIM), jnp.float32),
        ),
        mesh=mesh,
        scratch_types=(
            [pltpu.VMEM((CHUNK,), jnp.int32)] * (3 * NBUF)
            + [pltpu.VMEM((CHUNK, H), jnp.float32)] * (2 * NBUF)
            + [pltpu.VMEM((CHUNK, WIDTH_DIM), jnp.float32)] * NBUF
            + [pltpu.SemaphoreType.DMA] * (9 * NBUF)
        ),
    )


BM = 1024


def _mlp_block(xs_ref, xe_ref, xw_ref, wa_ref, wb_ref, ww_ref,
               b1_ref, w2_ref, b2_ref, out_ref):
    acc = jnp.dot(xs_ref[...].astype(jnp.bfloat16), wa_ref[...],
                  preferred_element_type=jnp.float32)
    acc += jnp.dot(xe_ref[...].astype(jnp.bfloat16), wb_ref[...],
                   preferred_element_type=jnp.float32)
    acc += jnp.dot(xw_ref[...].astype(jnp.bfloat16), ww_ref[...],
                   preferred_element_type=jnp.float32)
    acc += b1_ref[...]
    h = jnp.maximum(acc, 0.0).astype(jnp.bfloat16)
    out = jnp.dot(h, w2_ref[...], preferred_element_type=jnp.float32)
    out_ref[...] = out + b2_ref[...]


def _tc_mlp(xs, xe, xw, wa, wb, ww, b1, w2, b2):
    nrows = xs.shape[0]
    grid = (nrows // BM,)
    return pl.pallas_call(
        _mlp_block,
        grid=grid,
        in_specs=[
            pl.BlockSpec((BM, H), lambda i: (i, 0)),
            pl.BlockSpec((BM, H), lambda i: (i, 0)),
            pl.BlockSpec((BM, WIDTH_DIM), lambda i: (i, 0)),
            pl.BlockSpec((H, H), lambda i: (0, 0)),
            pl.BlockSpec((H, H), lambda i: (0, 0)),
            pl.BlockSpec((WIDTH_DIM, H), lambda i: (0, 0)),
            pl.BlockSpec((1, H), lambda i: (0, 0)),
            pl.BlockSpec((H, NUM_LABELS), lambda i: (0, 0)),
            pl.BlockSpec((1, NUM_LABELS), lambda i: (0, 0)),
        ],
        out_specs=pl.BlockSpec((BM, NUM_LABELS), lambda i: (i, 0)),
        out_shape=jax.ShapeDtypeStruct((nrows, NUM_LABELS), jnp.float32),
        compiler_params=pltpu.CompilerParams(
            dimension_semantics=("parallel",),
        ),
    )(xs, xe, xw, wa, wb, ww, b1, w2, b2)


def kernel(hidden_states, spans, width_table, W1, b1, W2, b2):
    hs_flat = hidden_states.reshape(B * S, H)

    offs = (jnp.arange(B, dtype=jnp.int32) * S)[:, None]
    idx_s = (spans[:, :, 0] + offs).reshape(NUM_ROWS)
    idx_e = (spans[:, :, 1] + offs).reshape(NUM_ROWS)
    idx_w = spans[:, :, 2].reshape(NUM_ROWS)

    wa = W1[:H].astype(jnp.bfloat16)
    wb = W1[H:2 * H].astype(jnp.bfloat16)
    ww = W1[2 * H:].astype(jnp.bfloat16)
    b1r = b1.reshape(1, H)
    w2 = W2.astype(jnp.bfloat16)
    b2r = b2.reshape(1, NUM_LABELS)

    sc_gather = _make_sc_gather(ROWS_PER_SPLIT)
    parts = []
    for c in range(NSPLIT):
        sl = slice(c * ROWS_PER_SPLIT, (c + 1) * ROWS_PER_SPLIT)
        gs, ge, gw = sc_gather(hs_flat, width_table,
                               idx_s[sl], idx_e[sl], idx_w[sl])
        parts.append(_tc_mlp(gs, ge, gw, wa, wb, ww, b1r, w2, b2r))

    logits = jnp.concatenate(parts, axis=0)
    return logits.reshape(B, N_SPANS, NUM_LABELS)
